# Initial kernel scaffold; baseline (speedup 1.0000x reference)
import jax
import jax.numpy as jnp
from jax.experimental import pallas as pl


def kernel(x, Wg, W1, b1, W2, b2):
    raise NotImplementedError("write your pallas kernel here")


# speedup vs baseline: 6.0260x; 6.0260x over previous
import functools

import jax
import jax.numpy as jnp
from jax import lax
from jax.experimental import pallas as pl
from jax.experimental.pallas import tpu as pltpu
from jax.experimental.pallas import tpu_sc as plsc

D_MODEL = 1024
D_FF = 4096
N_EXPERTS = 8
N_TOKENS = 4096
CAPACITY = int(N_TOKENS / N_EXPERTS * 1.25) + 1
C_PAD = 672
S = N_EXPERTS * C_PAD

SC_NC = 2
SC_NS = 16
SC_NW = SC_NC * SC_NS


def _gate_body(x_ref, wg_ref, top1_ref, w_ref):
    logits = lax.dot_general(
        x_ref[...], wg_ref[...], (((1,), (0,)), ((), ())),
        preferred_element_type=jnp.float32)
    m = jnp.max(logits, axis=-1, keepdims=True)
    e = jnp.exp(logits - m)
    w = jnp.max(e, axis=-1, keepdims=True) / jnp.sum(e, axis=-1, keepdims=True)
    lane = lax.broadcasted_iota(jnp.int32, logits.shape, 1)
    top1 = jnp.min(jnp.where(logits == m, lane, N_EXPERTS), axis=-1,
                   keepdims=True)
    top1_ref[...] = top1
    w_ref[...] = w


def _gating(x_flat, Wg):
    blk = 1024
    grid = N_TOKENS // blk
    top1, w = pl.pallas_call(
        _gate_body,
        grid=(grid,),
        in_specs=[
            pl.BlockSpec((blk, D_MODEL), lambda i: (i, 0)),
            pl.BlockSpec((D_MODEL, N_EXPERTS), lambda i: (0, 0)),
        ],
        out_specs=[
            pl.BlockSpec((blk, 1), lambda i: (i, 0)),
            pl.BlockSpec((blk, 1), lambda i: (i, 0)),
        ],
        out_shape=[
            jax.ShapeDtypeStruct((N_TOKENS, 1), jnp.int32),
            jax.ShapeDtypeStruct((N_TOKENS, 1), jnp.float32),
        ],
    )(x_flat, Wg)
    return top1[:, 0], w[:, 0]


@functools.lru_cache(maxsize=None)
def _make_sc_gather(n_rows, n_idx, chunk):
    assert n_idx % (SC_NW * chunk) == 0 and chunk % 8 == 0
    per_w = n_idx // SC_NW
    n_chunks = per_w // chunk
    mesh = plsc.VectorSubcoreMesh(core_axis_name="c", subcore_axis_name="s")

    @functools.partial(
        pl.kernel, mesh=mesh,
        out_type=jax.ShapeDtypeStruct((n_idx, D_MODEL), jnp.float32),
        scratch_types=[
            pltpu.VMEM((chunk,), jnp.int32),
            pltpu.VMEM((chunk, D_MODEL), jnp.float32),
            pltpu.SemaphoreType.DMA,
        ],
    )
    def gather_k(table_hbm, idx_hbm, out_hbm, idx_v, rows_v, sem):
        wid = lax.axis_index("s") * SC_NC + lax.axis_index("c")
        base = wid * per_w
        for c in range(n_chunks):
            off = base + c * chunk
            pltpu.sync_copy(idx_hbm.at[pl.ds(off, chunk)], idx_v)
            pltpu.async_copy(table_hbm.at[idx_v], rows_v, sem).wait()
            pltpu.sync_copy(rows_v, out_hbm.at[pl.ds(off, chunk)])

    return gather_k


def _gather_dispatch(table, idx):
    return _make_sc_gather(N_TOKENS, S, 24)(table, idx)


def _gather_combine(table, idx):
    return _make_sc_gather(S, N_TOKENS, 32)(table, idx)


_F_BLK = 512
_NF = D_FF // _F_BLK


def _ffn_body(xd_ref, w1_ref, b1_ref, w2_ref, b2_ref, wd_ref, out_ref):
    f = pl.program_id(1)
    x = xd_ref[0]
    h = jnp.dot(x, w1_ref[0], preferred_element_type=jnp.float32) + b1_ref[0]
    h = 0.5 * h * (1.0 + lax.erf(h * 0.7071067811865476))
    part = jnp.dot(h, w2_ref[0], preferred_element_type=jnp.float32)

    @pl.when(f == 0)
    def _():
        out_ref[0] = part

    @pl.when(f > 0)
    def _():
        out_ref[0] = out_ref[0] + part

    @pl.when(f == _NF - 1)
    def _():
        out_ref[0] = (out_ref[0] + b2_ref[0]) * wd_ref[0]


def _expert_ffn(xd, W1, b1, W2, b2, wd):
    return pl.pallas_call(
        _ffn_body,
        grid=(N_EXPERTS, _NF),
        in_specs=[
            pl.BlockSpec((1, C_PAD, D_MODEL), lambda e, f: (e, 0, 0)),
            pl.BlockSpec((1, D_MODEL, _F_BLK), lambda e, f: (e, 0, f)),
            pl.BlockSpec((1, 1, _F_BLK), lambda e, f: (e, 0, f)),
            pl.BlockSpec((1, _F_BLK, D_MODEL), lambda e, f: (e, f, 0)),
            pl.BlockSpec((1, 1, D_MODEL), lambda e, f: (e, 0, 0)),
            pl.BlockSpec((1, C_PAD, 1), lambda e, f: (e, 0, 0)),
        ],
        out_specs=pl.BlockSpec((1, C_PAD, D_MODEL), lambda e, f: (e, 0, 0)),
        out_shape=jax.ShapeDtypeStruct((N_EXPERTS, C_PAD, D_MODEL),
                                       jnp.float32),
        compiler_params=pltpu.CompilerParams(
            dimension_semantics=("parallel", "arbitrary")),
    )(xd, W1, b1.reshape(N_EXPERTS, 1, D_FF), W2,
      b2.reshape(N_EXPERTS, 1, D_MODEL), wd)


def kernel(x, Wg, W1, b1, W2, b2):
    B, T, D = x.shape
    x_flat = x.reshape(B * T, D)

    top1, w = _gating(x_flat, Wg)

    onehot = (top1[:, None] == jnp.arange(N_EXPERTS, dtype=jnp.int32)[None, :])
    pos = jnp.cumsum(onehot.astype(jnp.int32), axis=0)
    pos_i = jnp.take_along_axis(pos, top1[:, None], axis=1)[:, 0]
    kept = pos_i <= CAPACITY
    slot = top1 * C_PAD + (pos_i - 1)
    slot_or_oob = jnp.where(kept, slot, S)
    tok_ids = jnp.arange(N_TOKENS, dtype=jnp.int32)
    src = jnp.zeros((S,), jnp.int32).at[slot_or_oob].set(tok_ids, mode="drop")
    w_disp = jnp.zeros((S,), jnp.float32).at[slot_or_oob].set(w, mode="drop")
    combine_idx = jnp.where(kept, slot, C_PAD - 1).astype(jnp.int32)

    xd = _gather_dispatch(x_flat, src)
    y = _expert_ffn(xd.reshape(N_EXPERTS, C_PAD, D),
                    W1, b1, W2, b2, w_disp.reshape(N_EXPERTS, C_PAD, 1))
    out_flat = _gather_combine(y.reshape(S, D), combine_idx)
    return out_flat.reshape(B, T, D)
